# baseline (speedup 1.0000x reference)
import jax
import jax.numpy as jnp
from jax import lax
from jax.experimental import pallas as pl
from jax.experimental.pallas import tpu as pltpu
from jax.experimental.pallas import tpu_sc as plsc

N = 10000
DIN = 128
DH = 128
DOUT = 47
EPS = 1e-5

N_PAD = 10240
DHALF = DH // 2
BM = 512
EB = 512
NC = 2
NS = 16
ROWS_PER_TILE = N_PAD // NS


def _make_agg(rb_per_worker: int, with_cnt: bool):
    mesh = plsc.VectorSubcoreMesh(core_axis_name="c", subcore_axis_name="s")
    out_type = [jax.ShapeDtypeStruct((NC, N_PAD, DHALF), jnp.float32)]
    scratch = [
        pltpu.VMEM((EB,), jnp.int32),
        pltpu.VMEM((EB,), jnp.int32),
        pltpu.VMEM((EB,), jnp.int32),
        pltpu.VMEM((EB,), jnp.int32),
        pltpu.VMEM((EB, DHALF), jnp.float32),
        pltpu.VMEM((EB, DHALF), jnp.float32),
        pltpu.VMEM_SHARED((N_PAD, DHALF), jnp.float32),
        pltpu.SemaphoreType.DMA,
        pltpu.SemaphoreType.DMA,
        pltpu.SemaphoreType.DMA,
        pltpu.SemaphoreType.DMA,
        pltpu.SemaphoreType.DMA,
        pltpu.SemaphoreType.DMA,
    ]
    if with_cnt:
        out_type.append(jax.ShapeDtypeStruct((NC, N_PAD, 16), jnp.float32))
        scratch += [
            pltpu.VMEM((EB, 16), jnp.float32),
            pltpu.VMEM_SHARED((N_PAD, 16), jnp.float32),
        ]

    n_iters = rb_per_worker // 2

    def body(y_hbm, src_hbm, dst_hbm, zeros_hbm, zeros16_hbm, ones_hbm,
             acc_out, *rest):
        if with_cnt:
            (cnt_out, srcP, dstP, srcQ, dstQ, bufA, bufB, acc_sh,
             semGA, semGB, semSA, semSB, semIP, semIQ, ones_v, cnt_sh) = rest
        else:
            (srcP, dstP, srcQ, dstQ, bufA, bufB, acc_sh,
             semGA, semGB, semSA, semSB, semIP, semIQ) = rest
        c = lax.axis_index("c")
        s = lax.axis_index("s")
        row0 = s * ROWS_PER_TILE
        tile_base = s * rb_per_worker

        pltpu.sync_copy(zeros_hbm, acc_sh.at[pl.ds(row0, ROWS_PER_TILE)])
        if with_cnt:
            pltpu.sync_copy(zeros16_hbm, cnt_sh.at[pl.ds(row0, ROWS_PER_TILE)])
            pltpu.sync_copy(ones_hbm, ones_v)
        plsc.subcore_barrier()

        def load_idx(sbuf, dbuf, row, sem):
            return [
                pltpu.async_copy(src_hbm.at[c].at[row], sbuf, sem),
                pltpu.async_copy(dst_hbm.at[row], dbuf, sem),
            ]

        def fire_gathers(sbuf, rowbuf, sem):
            return [pltpu.async_copy(y_hbm.at[sbuf], rowbuf, sem)]

        def fire_scatters(dbuf, rowbuf, sem):
            return [pltpu.async_copy(rowbuf, acc_sh.at[dbuf], sem, add=True)]

        def cnt_adds(dbuf, pred):
            if not with_cnt:
                return

            @pl.when(pred)
            def _():
                pltpu.sync_copy(ones_v, cnt_sh.at[dbuf], add=True)

        def drain(descs):
            for d in descs:
                d.wait()

        @pl.loop(0, n_iters)
        def _(m):
            pred = (c == 0) == (m < n_iters // 2)
            base = tile_base + m * 2
            iP = load_idx(srcP, dstP, base, semIP)
            iQ = load_idx(srcQ, dstQ, base + 1, semIQ)
            drain(iP)
            gA = fire_gathers(srcP, bufA, semGA)
            drain(iQ)
            gB = fire_gathers(srcQ, bufB, semGB)
            drain(gA)
            sA = fire_scatters(dstP, bufA, semSA)
            cnt_adds(dstP, pred)
            drain(gB)
            sB = fire_scatters(dstQ, bufB, semSB)
            cnt_adds(dstQ, pred)
            drain(sA)
            drain(sB)

        plsc.subcore_barrier()
        sl = pl.ds(row0, ROWS_PER_TILE)
        pltpu.sync_copy(acc_sh.at[sl], acc_out.at[c].at[sl])
        if with_cnt:
            pltpu.sync_copy(cnt_sh.at[sl], cnt_out.at[c].at[sl])

    return pl.kernel(body, out_type=tuple(out_type), mesh=mesh,
                     scratch_types=tuple(scratch),
                     compiler_params=pltpu.CompilerParams(
                         use_tc_tiling_on_sc=False))


def _mm_body(x_ref, w_ref, o_ref):
    o_ref[...] = jnp.dot(x_ref[...], w_ref[...],
                         preferred_element_type=jnp.float32)


def _matmul(x, w):
    n, k = x.shape
    m = w.shape[1]
    return pl.pallas_call(
        _mm_body,
        grid=(n // BM,),
        in_specs=[pl.BlockSpec((BM, k), lambda i: (i, 0)),
                  pl.BlockSpec((k, m), lambda i: (0, 0))],
        out_specs=pl.BlockSpec((BM, m), lambda i: (i, 0)),
        out_shape=jax.ShapeDtypeStruct((n, m), jnp.float32),
    )(x, w)


def _combine_body(acc_ref, cnt_ref, x_ref, wr_ref, wn_ref, bl_ref, g_ref,
                  beta_ref, h_ref, y_ref):
    cnt = cnt_ref[0, :, 0:1] + cnt_ref[1, :, 0:1]
    inv = 1.0 / jnp.maximum(cnt, 1.0)
    agg = jnp.concatenate([acc_ref[0], acc_ref[1]], axis=1)
    h = agg * inv + bl_ref[...] + jnp.dot(
        x_ref[...], wr_ref[...], preferred_element_type=jnp.float32)
    scale = g_ref[...] * (1.0 / jnp.sqrt(1.0 + EPS))
    h = jnp.maximum(scale * h + beta_ref[...], 0.0)
    h_ref[...] = h
    y_ref[...] = jnp.dot(h, wn_ref[...], preferred_element_type=jnp.float32)


def _combine(acc, cnt, x, wrT, wnextT, bl, g, beta):
    return pl.pallas_call(
        _combine_body,
        grid=(N_PAD // BM,),
        in_specs=[
            pl.BlockSpec((NC, BM, DHALF), lambda i: (0, i, 0)),
            pl.BlockSpec((NC, BM, 16), lambda i: (0, i, 0)),
            pl.BlockSpec((BM, DH), lambda i: (i, 0)),
            pl.BlockSpec((DH, DH), lambda i: (0, 0)),
            pl.BlockSpec((DH, DH), lambda i: (0, 0)),
            pl.BlockSpec((1, DH), lambda i: (0, 0)),
            pl.BlockSpec((1, DH), lambda i: (0, 0)),
            pl.BlockSpec((1, DH), lambda i: (0, 0)),
        ],
        out_specs=[pl.BlockSpec((BM, DH), lambda i: (i, 0)),
                   pl.BlockSpec((BM, DH), lambda i: (i, 0))],
        out_shape=[jax.ShapeDtypeStruct((N_PAD, DH), jnp.float32),
                   jax.ShapeDtypeStruct((N_PAD, DH), jnp.float32)],
    )(acc, cnt, x, wrT, wnextT, bl, g, beta)


def _final_body(acc_ref, cnt_ref, h_ref, wr_ref, bl_ref, o_ref):
    cnt = cnt_ref[0, :, 0:1] + cnt_ref[1, :, 0:1]
    inv = 1.0 / jnp.maximum(cnt, 1.0)
    agg = jnp.concatenate([acc_ref[0], acc_ref[1]], axis=1)
    z = agg * inv + bl_ref[...] + jnp.dot(
        h_ref[...], wr_ref[...], preferred_element_type=jnp.float32)
    col = lax.broadcasted_iota(jnp.int32, z.shape, 1)
    valid = col < DOUT
    zm = jnp.where(valid, z, -jnp.inf)
    m = jnp.max(zm, axis=1, keepdims=True)
    e = jnp.where(valid, jnp.exp(z - m), 0.0)
    lse = jnp.log(jnp.sum(e, axis=1, keepdims=True))
    o_ref[...] = z - m - lse


def _final(acc, cnt, h, wrT, bl):
    return pl.pallas_call(
        _final_body,
        grid=(N_PAD // BM,),
        in_specs=[
            pl.BlockSpec((NC, BM, DHALF), lambda i: (0, i, 0)),
            pl.BlockSpec((NC, BM, 16), lambda i: (0, i, 0)),
            pl.BlockSpec((BM, DH), lambda i: (i, 0)),
            pl.BlockSpec((DH, DH), lambda i: (0, 0)),
            pl.BlockSpec((1, DH), lambda i: (0, 0)),
        ],
        out_specs=pl.BlockSpec((BM, DH), lambda i: (i, 0)),
        out_shape=jax.ShapeDtypeStruct((N_PAD, DH), jnp.float32),
    )(acc, cnt, h, wrT, bl)


def _split_cols(y):
    return jnp.concatenate([y[:, :DHALF], y[:, DHALF:]], axis=0)


def kernel(x, edge_index, Wl1, bl1, Wr1, g1, b1, Wl2, bl2, Wr2, g2, b2,
           Wl3, bl3, Wr3):
    e = edge_index.shape[1]
    blk_edges = NS * EB * 2
    e_pad = ((e + blk_edges - 1) // blk_edges) * blk_edges
    rb_total = e_pad // EB
    rb_per_worker = rb_total // NS

    src1 = jnp.concatenate(
        [edge_index[0], jnp.zeros((e_pad - e,), jnp.int32)]).reshape(rb_total, EB)
    src = jnp.stack([src1, src1 + N_PAD])
    dst = jnp.concatenate(
        [edge_index[1], jnp.full((e_pad - e,), N, jnp.int32)]).reshape(rb_total, EB)

    x_p = jnp.concatenate([x, jnp.zeros((N_PAD - N, DIN), jnp.float32)])
    zeros_d = jnp.zeros((ROWS_PER_TILE, DHALF), jnp.float32)
    zeros16 = jnp.zeros((ROWS_PER_TILE, 16), jnp.float32)
    ones16 = jnp.ones((EB, 16), jnp.float32)

    wl3T = jnp.zeros((DH, DH), jnp.float32).at[:, :DOUT].set(Wl3.T)
    wr3T = jnp.zeros((DH, DH), jnp.float32).at[:, :DOUT].set(Wr3.T)
    bl3p = jnp.zeros((1, DH), jnp.float32).at[0, :DOUT].set(bl3)

    agg_cnt = _make_agg(rb_per_worker, True)
    agg = _make_agg(rb_per_worker, False)

    r2 = lambda v: v.reshape(1, DH)

    y1 = _matmul(x_p, Wl1.T)
    acc1, cnt = agg_cnt(_split_cols(y1), src, dst, zeros_d, zeros16, ones16)
    h1, y2 = _combine(acc1, cnt, x_p, Wr1.T, Wl2.T, r2(bl1), r2(g1), r2(b1))
    (acc2,) = agg(_split_cols(y2), src, dst, zeros_d, zeros16, ones16)
    h2, y3 = _combine(acc2, cnt, h1, Wr2.T, wl3T, r2(bl2), r2(g2), r2(b2))
    (acc3,) = agg(_split_cols(y3), src, dst, zeros_d, zeros16, ones16)
    o = _final(acc3, cnt, h2, wr3T, bl3p)
    return o[:N, :DOUT]


# speedup vs baseline: 1.0313x; 1.0313x over previous
import jax
import jax.numpy as jnp
from jax import lax
from jax.experimental import pallas as pl
from jax.experimental.pallas import tpu as pltpu
from jax.experimental.pallas import tpu_sc as plsc

N = 10000
DIN = 128
DH = 128
DOUT = 47
EPS = 1e-5

N_PAD = 10240
DHALF = DH // 2
BM = 512
EB = 256
NBUF = 4
NC = 2
NS = 16
ROWS_PER_TILE = N_PAD // NS


def _make_agg(rb_per_worker: int, with_cnt: bool):
    mesh = plsc.VectorSubcoreMesh(core_axis_name="c", subcore_axis_name="s")
    out_type = [jax.ShapeDtypeStruct((NC, N_PAD, DHALF), jnp.float32)]
    scratch = (
        [pltpu.VMEM((EB,), jnp.int32) for _ in range(NBUF)]
        + [pltpu.VMEM((EB,), jnp.int32) for _ in range(NBUF)]
        + [pltpu.VMEM((EB, DHALF), jnp.float32) for _ in range(NBUF)]
        + [pltpu.VMEM_SHARED((N_PAD, DHALF), jnp.float32)]
        + [pltpu.SemaphoreType.DMA for _ in range(3 * NBUF)]
    )
    if with_cnt:
        out_type.append(jax.ShapeDtypeStruct((NC, N_PAD, 16), jnp.float32))
        scratch += [
            pltpu.VMEM((EB, 16), jnp.float32),
            pltpu.VMEM_SHARED((N_PAD, 16), jnp.float32),
        ]

    n_iters = rb_per_worker // NBUF

    def body(y_hbm, src_hbm, dst_hbm, zeros_hbm, zeros16_hbm, ones_hbm,
             acc_out, *rest):
        rest = list(rest)
        cnt_out = rest.pop(0) if with_cnt else None
        srcs = [rest.pop(0) for _ in range(NBUF)]
        dsts = [rest.pop(0) for _ in range(NBUF)]
        bufs = [rest.pop(0) for _ in range(NBUF)]
        acc_sh = rest.pop(0)
        semG = [rest.pop(0) for _ in range(NBUF)]
        semS = [rest.pop(0) for _ in range(NBUF)]
        semI = [rest.pop(0) for _ in range(NBUF)]
        if with_cnt:
            ones_v, cnt_sh = rest
        c = lax.axis_index("c")
        s = lax.axis_index("s")
        row0 = s * ROWS_PER_TILE
        tile_base = s * rb_per_worker

        pltpu.sync_copy(zeros_hbm, acc_sh.at[pl.ds(row0, ROWS_PER_TILE)])
        if with_cnt:
            pltpu.sync_copy(zeros16_hbm, cnt_sh.at[pl.ds(row0, ROWS_PER_TILE)])
            pltpu.sync_copy(ones_hbm, ones_v)
        plsc.subcore_barrier()

        def load_idx(sbuf, dbuf, row, sem):
            return [
                pltpu.async_copy(src_hbm.at[c].at[row], sbuf, sem),
                pltpu.async_copy(dst_hbm.at[row], dbuf, sem),
            ]

        def fire_gathers(sbuf, rowbuf, sem):
            return [pltpu.async_copy(y_hbm.at[sbuf], rowbuf, sem)]

        def fire_scatters(dbuf, rowbuf, sem):
            return [pltpu.async_copy(rowbuf, acc_sh.at[dbuf], sem, add=True)]

        def cnt_adds(dbuf, pred):
            if not with_cnt:
                return

            @pl.when(pred)
            def _():
                pltpu.sync_copy(ones_v, cnt_sh.at[dbuf], add=True)

        def drain(descs):
            for d in descs:
                d.wait()

        @pl.loop(0, n_iters)
        def _(m):
            pred = (c == 0) == (m < n_iters // 2)
            base = tile_base + m * NBUF
            idx = [load_idx(srcs[k], dsts[k], base + k, semI[k])
                   for k in range(NBUF)]
            g = []
            drain(idx[0])
            g.append(fire_gathers(srcs[0], bufs[0], semG[0]))
            drain(idx[1])
            g.append(fire_gathers(srcs[1], bufs[1], semG[1]))
            sc = []
            for k in range(NBUF):
                drain(g[k])
                sc.append(fire_scatters(dsts[k], bufs[k], semS[k]))
                if k + 2 < NBUF:
                    drain(idx[k + 2])
                    g.append(fire_gathers(srcs[k + 2], bufs[k + 2],
                                          semG[k + 2]))
                cnt_adds(dsts[k], pred)
            for k in range(NBUF):
                drain(sc[k])

        plsc.subcore_barrier()
        sl = pl.ds(row0, ROWS_PER_TILE)
        pltpu.sync_copy(acc_sh.at[sl], acc_out.at[c].at[sl])
        if with_cnt:
            pltpu.sync_copy(cnt_sh.at[sl], cnt_out.at[c].at[sl])

    return pl.kernel(body, out_type=tuple(out_type), mesh=mesh,
                     scratch_types=tuple(scratch),
                     compiler_params=pltpu.CompilerParams(
                         use_tc_tiling_on_sc=False))


def _mm_body(x_ref, w_ref, o_ref):
    o_ref[...] = jnp.dot(x_ref[...], w_ref[...],
                         preferred_element_type=jnp.float32)


def _matmul(x, w):
    n, k = x.shape
    m = w.shape[1]
    return pl.pallas_call(
        _mm_body,
        grid=(n // BM,),
        in_specs=[pl.BlockSpec((BM, k), lambda i: (i, 0)),
                  pl.BlockSpec((k, m), lambda i: (0, 0))],
        out_specs=pl.BlockSpec((BM, m), lambda i: (i, 0)),
        out_shape=jax.ShapeDtypeStruct((n, m), jnp.float32),
    )(x, w)


def _combine_body(acc_ref, cnt_ref, x_ref, wr_ref, wn_ref, bl_ref, g_ref,
                  beta_ref, h_ref, y_ref):
    cnt = cnt_ref[0, :, 0:1] + cnt_ref[1, :, 0:1]
    inv = 1.0 / jnp.maximum(cnt, 1.0)
    agg = jnp.concatenate([acc_ref[0], acc_ref[1]], axis=1)
    h = agg * inv + bl_ref[...] + jnp.dot(
        x_ref[...], wr_ref[...], preferred_element_type=jnp.float32)
    scale = g_ref[...] * (1.0 / jnp.sqrt(1.0 + EPS))
    h = jnp.maximum(scale * h + beta_ref[...], 0.0)
    h_ref[...] = h
    y_ref[...] = jnp.dot(h, wn_ref[...], preferred_element_type=jnp.float32)


def _combine(acc, cnt, x, wrT, wnextT, bl, g, beta):
    return pl.pallas_call(
        _combine_body,
        grid=(N_PAD // BM,),
        in_specs=[
            pl.BlockSpec((NC, BM, DHALF), lambda i: (0, i, 0)),
            pl.BlockSpec((NC, BM, 16), lambda i: (0, i, 0)),
            pl.BlockSpec((BM, DH), lambda i: (i, 0)),
            pl.BlockSpec((DH, DH), lambda i: (0, 0)),
            pl.BlockSpec((DH, DH), lambda i: (0, 0)),
            pl.BlockSpec((1, DH), lambda i: (0, 0)),
            pl.BlockSpec((1, DH), lambda i: (0, 0)),
            pl.BlockSpec((1, DH), lambda i: (0, 0)),
        ],
        out_specs=[pl.BlockSpec((BM, DH), lambda i: (i, 0)),
                   pl.BlockSpec((BM, DH), lambda i: (i, 0))],
        out_shape=[jax.ShapeDtypeStruct((N_PAD, DH), jnp.float32),
                   jax.ShapeDtypeStruct((N_PAD, DH), jnp.float32)],
    )(acc, cnt, x, wrT, wnextT, bl, g, beta)


def _final_body(acc_ref, cnt_ref, h_ref, wr_ref, bl_ref, o_ref):
    cnt = cnt_ref[0, :, 0:1] + cnt_ref[1, :, 0:1]
    inv = 1.0 / jnp.maximum(cnt, 1.0)
    agg = jnp.concatenate([acc_ref[0], acc_ref[1]], axis=1)
    z = agg * inv + bl_ref[...] + jnp.dot(
        h_ref[...], wr_ref[...], preferred_element_type=jnp.float32)
    col = lax.broadcasted_iota(jnp.int32, z.shape, 1)
    valid = col < DOUT
    zm = jnp.where(valid, z, -jnp.inf)
    m = jnp.max(zm, axis=1, keepdims=True)
    e = jnp.where(valid, jnp.exp(z - m), 0.0)
    lse = jnp.log(jnp.sum(e, axis=1, keepdims=True))
    o_ref[...] = z - m - lse


def _final(acc, cnt, h, wrT, bl):
    return pl.pallas_call(
        _final_body,
        grid=(N_PAD // BM,),
        in_specs=[
            pl.BlockSpec((NC, BM, DHALF), lambda i: (0, i, 0)),
            pl.BlockSpec((NC, BM, 16), lambda i: (0, i, 0)),
            pl.BlockSpec((BM, DH), lambda i: (i, 0)),
            pl.BlockSpec((DH, DH), lambda i: (0, 0)),
            pl.BlockSpec((1, DH), lambda i: (0, 0)),
        ],
        out_specs=pl.BlockSpec((BM, DH), lambda i: (i, 0)),
        out_shape=jax.ShapeDtypeStruct((N_PAD, DH), jnp.float32),
    )(acc, cnt, h, wrT, bl)


def _split_cols(y):
    return jnp.concatenate([y[:, :DHALF], y[:, DHALF:]], axis=0)


def kernel(x, edge_index, Wl1, bl1, Wr1, g1, b1, Wl2, bl2, Wr2, g2, b2,
           Wl3, bl3, Wr3):
    e = edge_index.shape[1]
    blk_edges = NS * EB * NBUF
    e_pad = ((e + blk_edges - 1) // blk_edges) * blk_edges
    rb_total = e_pad // EB
    rb_per_worker = rb_total // NS

    src1 = jnp.concatenate(
        [edge_index[0], jnp.zeros((e_pad - e,), jnp.int32)]).reshape(rb_total, EB)
    src = jnp.stack([src1, src1 + N_PAD])
    dst = jnp.concatenate(
        [edge_index[1], jnp.full((e_pad - e,), N, jnp.int32)]).reshape(rb_total, EB)

    x_p = jnp.concatenate([x, jnp.zeros((N_PAD - N, DIN), jnp.float32)])
    zeros_d = jnp.zeros((ROWS_PER_TILE, DHALF), jnp.float32)
    zeros16 = jnp.zeros((ROWS_PER_TILE, 16), jnp.float32)
    ones16 = jnp.ones((EB, 16), jnp.float32)

    wl3T = jnp.zeros((DH, DH), jnp.float32).at[:, :DOUT].set(Wl3.T)
    wr3T = jnp.zeros((DH, DH), jnp.float32).at[:, :DOUT].set(Wr3.T)
    bl3p = jnp.zeros((1, DH), jnp.float32).at[0, :DOUT].set(bl3)

    agg_cnt = _make_agg(rb_per_worker, True)
    agg = _make_agg(rb_per_worker, False)

    r2 = lambda v: v.reshape(1, DH)

    y1 = _matmul(x_p, Wl1.T)
    acc1, cnt = agg_cnt(_split_cols(y1), src, dst, zeros_d, zeros16, ones16)
    h1, y2 = _combine(acc1, cnt, x_p, Wr1.T, Wl2.T, r2(bl1), r2(g1), r2(b1))
    (acc2,) = agg(_split_cols(y2), src, dst, zeros_d, zeros16, ones16)
    h2, y3 = _combine(acc2, cnt, h1, Wr2.T, wl3T, r2(bl2), r2(g2), r2(b2))
    (acc3,) = agg(_split_cols(y3), src, dst, zeros_d, zeros16, ones16)
    o = _final(acc3, cnt, h2, wr3T, bl3p)
    return o[:N, :DOUT]


# speedup vs baseline: 1.2445x; 1.2067x over previous
import jax
import jax.numpy as jnp
from jax import lax
from jax.experimental import pallas as pl
from jax.experimental.pallas import tpu as pltpu
from jax.experimental.pallas import tpu_sc as plsc

N = 10000
DIN = 128
DH = 128
DOUT = 47
EPS = 1e-5

N_PAD = 10240
DHALF = DH // 2
BM = 512
EB = 256
NBUF = 4
NC = 2
NS = 16
ROWS_PER_TILE = N_PAD // NS


def _make_agg(rb_per_worker: int, with_cnt: bool, dh: int):
    mesh = plsc.VectorSubcoreMesh(core_axis_name="c", subcore_axis_name="s")
    out_type = [jax.ShapeDtypeStruct((NC, N_PAD, dh), jnp.float32)]
    scratch = (
        [pltpu.VMEM((EB,), jnp.int32) for _ in range(NBUF)]
        + [pltpu.VMEM((EB,), jnp.int32) for _ in range(NBUF)]
        + [pltpu.VMEM((EB, dh), jnp.float32) for _ in range(NBUF)]
        + [pltpu.VMEM_SHARED((N_PAD, dh), jnp.float32)]
        + [pltpu.SemaphoreType.DMA for _ in range(3 * NBUF)]
    )
    if with_cnt:
        out_type.append(jax.ShapeDtypeStruct((NC, N_PAD, 16), jnp.float32))
        scratch += [
            pltpu.VMEM((EB, 16), jnp.float32),
            pltpu.VMEM_SHARED((N_PAD, 16), jnp.float32),
        ]

    n_iters = rb_per_worker // NBUF

    def body(y_hbm, src_hbm, dst_hbm, zeros_hbm, zeros16_hbm, ones_hbm,
             acc_out, *rest):
        rest = list(rest)
        cnt_out = rest.pop(0) if with_cnt else None
        srcs = [rest.pop(0) for _ in range(NBUF)]
        dsts = [rest.pop(0) for _ in range(NBUF)]
        bufs = [rest.pop(0) for _ in range(NBUF)]
        acc_sh = rest.pop(0)
        semG = [rest.pop(0) for _ in range(NBUF)]
        semS = [rest.pop(0) for _ in range(NBUF)]
        semI = [rest.pop(0) for _ in range(NBUF)]
        if with_cnt:
            ones_v, cnt_sh = rest
        c = lax.axis_index("c")
        s = lax.axis_index("s")
        row0 = s * ROWS_PER_TILE
        tile_base = s * rb_per_worker

        pltpu.sync_copy(zeros_hbm, acc_sh.at[pl.ds(row0, ROWS_PER_TILE)])
        if with_cnt:
            pltpu.sync_copy(zeros16_hbm, cnt_sh.at[pl.ds(row0, ROWS_PER_TILE)])
            pltpu.sync_copy(ones_hbm, ones_v)
        plsc.subcore_barrier()

        def load_idx(sbuf, dbuf, row, sem):
            return [
                pltpu.async_copy(src_hbm.at[c].at[row], sbuf, sem),
                pltpu.async_copy(dst_hbm.at[row], dbuf, sem),
            ]

        def fire_gathers(sbuf, rowbuf, sem):
            return [pltpu.async_copy(y_hbm.at[sbuf], rowbuf, sem)]

        def fire_scatters(dbuf, rowbuf, sem):
            return [pltpu.async_copy(rowbuf, acc_sh.at[dbuf], sem, add=True)]

        def cnt_adds(dbuf, pred):
            if not with_cnt:
                return

            @pl.when(pred)
            def _():
                pltpu.sync_copy(ones_v, cnt_sh.at[dbuf], add=True)

        def drain(descs):
            for d in descs:
                d.wait()

        @pl.loop(0, n_iters)
        def _(m):
            pred = (c == 0) == (m < n_iters // 2)
            base = tile_base + m * NBUF
            idx = [load_idx(srcs[k], dsts[k], base + k, semI[k])
                   for k in range(NBUF)]
            g = []
            drain(idx[0])
            g.append(fire_gathers(srcs[0], bufs[0], semG[0]))
            drain(idx[1])
            g.append(fire_gathers(srcs[1], bufs[1], semG[1]))
            sc = []
            for k in range(NBUF):
                drain(g[k])
                sc.append(fire_scatters(dsts[k], bufs[k], semS[k]))
                if k + 2 < NBUF:
                    drain(idx[k + 2])
                    g.append(fire_gathers(srcs[k + 2], bufs[k + 2],
                                          semG[k + 2]))
                cnt_adds(dsts[k], pred)
            for k in range(NBUF):
                drain(sc[k])

        plsc.subcore_barrier()
        sl = pl.ds(row0, ROWS_PER_TILE)
        pltpu.sync_copy(acc_sh.at[sl], acc_out.at[c].at[sl])
        if with_cnt:
            pltpu.sync_copy(cnt_sh.at[sl], cnt_out.at[c].at[sl])

    return pl.kernel(body, out_type=tuple(out_type), mesh=mesh,
                     scratch_types=tuple(scratch),
                     compiler_params=pltpu.CompilerParams(
                         use_tc_tiling_on_sc=False))


def _mm_body(x_ref, w_ref, o_ref):
    o_ref[...] = jnp.dot(x_ref[...], w_ref[...],
                         preferred_element_type=jnp.float32)


def _matmul(x, w):
    n, k = x.shape
    m = w.shape[1]
    return pl.pallas_call(
        _mm_body,
        grid=(n // BM,),
        in_specs=[pl.BlockSpec((BM, k), lambda i: (i, 0)),
                  pl.BlockSpec((k, m), lambda i: (0, 0))],
        out_specs=pl.BlockSpec((BM, m), lambda i: (i, 0)),
        out_shape=jax.ShapeDtypeStruct((n, m), jnp.float32),
    )(x, w)


def _combine_body(acc_ref, cnt_ref, x_ref, wr_ref, wn_ref, bl_ref, g_ref,
                  beta_ref, h_ref, y_ref):
    cnt = cnt_ref[0, :, 0:1] + cnt_ref[1, :, 0:1]
    inv = 1.0 / jnp.maximum(cnt, 1.0)
    agg = jnp.concatenate([acc_ref[0], acc_ref[1]], axis=1)
    h = agg * inv + bl_ref[...] + jnp.dot(
        x_ref[...], wr_ref[...], preferred_element_type=jnp.float32)
    scale = g_ref[...] * (1.0 / jnp.sqrt(1.0 + EPS))
    h = jnp.maximum(scale * h + beta_ref[...], 0.0)
    h_ref[...] = h
    y = jnp.dot(h, wn_ref[...], preferred_element_type=jnp.float32)
    ynh = y_ref.shape[2]
    y_ref[0] = y[:, :ynh]
    y_ref[1] = y[:, ynh:]


def _combine(acc, cnt, x, wrT, wnextT, bl, g, beta):
    ah = acc.shape[2]
    ynh = wnextT.shape[1] // 2
    return pl.pallas_call(
        _combine_body,
        grid=(N_PAD // BM,),
        in_specs=[
            pl.BlockSpec((NC, BM, ah), lambda i: (0, i, 0)),
            pl.BlockSpec((NC, BM, 16), lambda i: (0, i, 0)),
            pl.BlockSpec((BM, DH), lambda i: (i, 0)),
            pl.BlockSpec((DH, DH), lambda i: (0, 0)),
            pl.BlockSpec((DH, wnextT.shape[1]), lambda i: (0, 0)),
            pl.BlockSpec((1, DH), lambda i: (0, 0)),
            pl.BlockSpec((1, DH), lambda i: (0, 0)),
            pl.BlockSpec((1, DH), lambda i: (0, 0)),
        ],
        out_specs=[pl.BlockSpec((BM, DH), lambda i: (i, 0)),
                   pl.BlockSpec((NC, BM, ynh), lambda i: (0, i, 0))],
        out_shape=[jax.ShapeDtypeStruct((N_PAD, DH), jnp.float32),
                   jax.ShapeDtypeStruct((NC, N_PAD, ynh), jnp.float32)],
    )(acc, cnt, x, wrT, wnextT, bl, g, beta)


D3 = 64


def _final_body(acc_ref, cnt_ref, h_ref, wr_ref, bl_ref, o_ref):
    cnt = cnt_ref[0, :, 0:1] + cnt_ref[1, :, 0:1]
    inv = 1.0 / jnp.maximum(cnt, 1.0)
    agg = jnp.concatenate([acc_ref[0], acc_ref[1]], axis=1)
    z = agg * inv + bl_ref[...] + jnp.dot(
        h_ref[...], wr_ref[...], preferred_element_type=jnp.float32)
    col = lax.broadcasted_iota(jnp.int32, z.shape, 1)
    valid = col < DOUT
    zm = jnp.where(valid, z, -jnp.inf)
    m = jnp.max(zm, axis=1, keepdims=True)
    e = jnp.where(valid, jnp.exp(z - m), 0.0)
    lse = jnp.log(jnp.sum(e, axis=1, keepdims=True))
    o_ref[...] = z - m - lse


def _final(acc, cnt, h, wrT, bl):
    return pl.pallas_call(
        _final_body,
        grid=(N_PAD // BM,),
        in_specs=[
            pl.BlockSpec((NC, BM, D3 // 2), lambda i: (0, i, 0)),
            pl.BlockSpec((NC, BM, 16), lambda i: (0, i, 0)),
            pl.BlockSpec((BM, DH), lambda i: (i, 0)),
            pl.BlockSpec((DH, D3), lambda i: (0, 0)),
            pl.BlockSpec((1, D3), lambda i: (0, 0)),
        ],
        out_specs=pl.BlockSpec((BM, D3), lambda i: (i, 0)),
        out_shape=jax.ShapeDtypeStruct((N_PAD, D3), jnp.float32),
    )(acc, cnt, h, wrT, bl)


def _split_cols(y):
    return jnp.concatenate([y[:, :DHALF], y[:, DHALF:]], axis=0)


def kernel(x, edge_index, Wl1, bl1, Wr1, g1, b1, Wl2, bl2, Wr2, g2, b2,
           Wl3, bl3, Wr3):
    e = edge_index.shape[1]
    blk_edges = NS * EB * NBUF
    e_pad = ((e + blk_edges - 1) // blk_edges) * blk_edges
    rb_total = e_pad // EB
    rb_per_worker = rb_total // NS

    src1 = jnp.concatenate(
        [edge_index[0], jnp.zeros((e_pad - e,), jnp.int32)]).reshape(rb_total, EB)
    src = jnp.stack([src1, src1 + N_PAD])
    dst = jnp.concatenate(
        [edge_index[1], jnp.full((e_pad - e,), N, jnp.int32)]).reshape(rb_total, EB)

    x_p = jnp.concatenate([x, jnp.zeros((N_PAD - N, DIN), jnp.float32)])
    zeros_d = jnp.zeros((ROWS_PER_TILE, DHALF), jnp.float32)
    zeros16 = jnp.zeros((ROWS_PER_TILE, 16), jnp.float32)
    ones16 = jnp.ones((EB, 16), jnp.float32)

    wl3T = jnp.zeros((DH, D3), jnp.float32).at[:, :DOUT].set(Wl3.T)
    wr3T = jnp.zeros((DH, D3), jnp.float32).at[:, :DOUT].set(Wr3.T)
    bl3p = jnp.zeros((1, D3), jnp.float32).at[0, :DOUT].set(bl3)

    agg_cnt = _make_agg(rb_per_worker, True, DHALF)
    agg2 = _make_agg(rb_per_worker, False, DHALF)
    agg3 = _make_agg(rb_per_worker, False, D3 // 2)
    zeros_d3 = jnp.zeros((ROWS_PER_TILE, D3 // 2), jnp.float32)

    r2 = lambda v: v.reshape(1, DH)

    y1 = _matmul(x_p, Wl1.T)
    acc1, cnt = agg_cnt(_split_cols(y1), src, dst, zeros_d, zeros16, ones16)
    h1, y2 = _combine(acc1, cnt, x_p, Wr1.T, Wl2.T, r2(bl1), r2(g1), r2(b1))
    (acc2,) = agg2(y2.reshape(NC * N_PAD, DHALF), src, dst,
                   zeros_d, zeros16, ones16)
    h2, y3 = _combine(acc2, cnt, h1, Wr2.T, wl3T, r2(bl2), r2(g2), r2(b2))
    (acc3,) = agg3(y3.reshape(NC * N_PAD, D3 // 2), src, dst,
                   zeros_d3, zeros16, ones16)
    o = _final(acc3, cnt, h2, wr3T, bl3p)
    return o[:N, :DOUT]


# speedup vs baseline: 2.1596x; 1.7353x over previous
import jax
import jax.numpy as jnp
from jax import lax
from jax.experimental import pallas as pl
from jax.experimental.pallas import tpu as pltpu
from jax.experimental.pallas import tpu_sc as plsc

N = 10000
DIN = 128
DH = 128
DOUT = 47
EPS = 1e-5

N_PAD = 10240
DHALF = DH // 2
BM = 512
EB = 320
NBUF = 8
PRE = 4
NC = 2
NS = 16
ROWS_PER_TILE = N_PAD // NS


def _make_agg(rb_per_worker: int, with_cnt: bool, dh: int):
    mesh = plsc.VectorSubcoreMesh(core_axis_name="c", subcore_axis_name="s")
    out_type = [jax.ShapeDtypeStruct((NC, N_PAD, dh), jnp.bfloat16)]
    scratch = (
        [pltpu.VMEM((EB,), jnp.int32) for _ in range(NBUF)]
        + [pltpu.VMEM((EB,), jnp.int32) for _ in range(NBUF)]
        + [pltpu.VMEM((EB, dh), jnp.bfloat16) for _ in range(NBUF)]
        + [pltpu.VMEM_SHARED((N_PAD, dh), jnp.bfloat16)]
        + [pltpu.SemaphoreType.DMA for _ in range(3 * NBUF)]
    )
    if with_cnt:
        out_type.append(jax.ShapeDtypeStruct((NC, N_PAD, 16), jnp.float32))
        scratch += [
            pltpu.VMEM((EB, 16), jnp.float32),
            pltpu.VMEM_SHARED((N_PAD, 16), jnp.float32),
        ]

    n_iters = rb_per_worker // NBUF

    def body(y_hbm, src_hbm, dst_hbm, zeros_hbm, zeros16_hbm, ones_hbm,
             acc_out, *rest):
        rest = list(rest)
        cnt_out = rest.pop(0) if with_cnt else None
        srcs = [rest.pop(0) for _ in range(NBUF)]
        dsts = [rest.pop(0) for _ in range(NBUF)]
        bufs = [rest.pop(0) for _ in range(NBUF)]
        acc_sh = rest.pop(0)
        semG = [rest.pop(0) for _ in range(NBUF)]
        semS = [rest.pop(0) for _ in range(NBUF)]
        semI = [rest.pop(0) for _ in range(NBUF)]
        if with_cnt:
            ones_v, cnt_sh = rest
        c = lax.axis_index("c")
        s = lax.axis_index("s")
        row0 = s * ROWS_PER_TILE
        tile_base = s * rb_per_worker

        pltpu.sync_copy(zeros_hbm, acc_sh.at[pl.ds(row0, ROWS_PER_TILE)])
        if with_cnt:
            pltpu.sync_copy(zeros16_hbm, cnt_sh.at[pl.ds(row0, ROWS_PER_TILE)])
            pltpu.sync_copy(ones_hbm, ones_v)
        plsc.subcore_barrier()

        def load_idx(sbuf, dbuf, row, sem):
            return [
                pltpu.async_copy(src_hbm.at[row], sbuf, sem),
                pltpu.async_copy(dst_hbm.at[row], dbuf, sem),
            ]

        def fire_gathers(sbuf, rowbuf, sem):
            return [pltpu.async_copy(y_hbm.at[c].at[sbuf], rowbuf, sem)]

        def fire_scatters(dbuf, rowbuf, sem):
            return [pltpu.async_copy(rowbuf, acc_sh.at[dbuf], sem, add=True)]

        def cnt_adds(dbuf, pred):
            if not with_cnt:
                return

            @pl.when(pred)
            def _():
                pltpu.sync_copy(ones_v, cnt_sh.at[dbuf], add=True)

        def drain(descs):
            for d in descs:
                d.wait()

        @pl.loop(0, n_iters)
        def _(m):
            pred = (c == 0) == (m < n_iters // 2)
            base = tile_base + m * NBUF
            idx = [load_idx(srcs[k], dsts[k], base + k, semI[k])
                   for k in range(NBUF)]
            g = []
            for k in range(PRE):
                drain(idx[k])
                g.append(fire_gathers(srcs[k], bufs[k], semG[k]))
            sc = []
            for k in range(NBUF):
                drain(g[k])
                sc.append(fire_scatters(dsts[k], bufs[k], semS[k]))
                if k + PRE < NBUF:
                    drain(idx[k + PRE])
                    g.append(fire_gathers(srcs[k + PRE], bufs[k + PRE],
                                          semG[k + PRE]))
                cnt_adds(dsts[k], pred)
            for k in range(NBUF):
                drain(sc[k])

        plsc.subcore_barrier()
        sl = pl.ds(row0, ROWS_PER_TILE)
        pltpu.sync_copy(acc_sh.at[sl], acc_out.at[c].at[sl])
        if with_cnt:
            pltpu.sync_copy(cnt_sh.at[sl], cnt_out.at[c].at[sl])

    return pl.kernel(body, out_type=tuple(out_type), mesh=mesh,
                     scratch_types=tuple(scratch),
                     compiler_params=pltpu.CompilerParams(
                         use_tc_tiling_on_sc=False))


def _mm_body(x_ref, w_ref, o_ref):
    y = jnp.dot(x_ref[...], w_ref[...],
                preferred_element_type=jnp.float32).astype(jnp.bfloat16)
    ynh = o_ref.shape[2]
    o_ref[0] = y[:, :ynh]
    o_ref[1] = y[:, ynh:]


def _matmul(x, w):
    n, k = x.shape
    m = w.shape[1]
    return pl.pallas_call(
        _mm_body,
        grid=(n // BM,),
        in_specs=[pl.BlockSpec((BM, k), lambda i: (i, 0)),
                  pl.BlockSpec((k, m), lambda i: (0, 0))],
        out_specs=pl.BlockSpec((NC, BM, m // 2), lambda i: (0, i, 0)),
        out_shape=jax.ShapeDtypeStruct((NC, n, m // 2), jnp.bfloat16),
    )(x, w)


def _combine_body(acc_ref, cnt_ref, x_ref, wr_ref, wn_ref, bl_ref, g_ref,
                  beta_ref, h_ref, y_ref):
    cnt = cnt_ref[0, :, 0:1] + cnt_ref[1, :, 0:1]
    inv = 1.0 / jnp.maximum(cnt, 1.0)
    agg = jnp.concatenate([acc_ref[0], acc_ref[1]], axis=1).astype(jnp.float32)
    h = agg * inv + bl_ref[...] + jnp.dot(
        x_ref[...], wr_ref[...], preferred_element_type=jnp.float32)
    scale = g_ref[...] * (1.0 / jnp.sqrt(1.0 + EPS))
    h = jnp.maximum(scale * h + beta_ref[...], 0.0)
    h_ref[...] = h
    y = jnp.dot(h, wn_ref[...],
                preferred_element_type=jnp.float32).astype(jnp.bfloat16)
    ynh = y_ref.shape[2]
    y_ref[0] = y[:, :ynh]
    y_ref[1] = y[:, ynh:]


def _combine(acc, cnt, x, wrT, wnextT, bl, g, beta):
    ah = acc.shape[2]
    ynh = wnextT.shape[1] // 2
    return pl.pallas_call(
        _combine_body,
        grid=(N_PAD // BM,),
        in_specs=[
            pl.BlockSpec((NC, BM, ah), lambda i: (0, i, 0)),
            pl.BlockSpec((NC, BM, 16), lambda i: (0, i, 0)),
            pl.BlockSpec((BM, DH), lambda i: (i, 0)),
            pl.BlockSpec((DH, DH), lambda i: (0, 0)),
            pl.BlockSpec((DH, wnextT.shape[1]), lambda i: (0, 0)),
            pl.BlockSpec((1, DH), lambda i: (0, 0)),
            pl.BlockSpec((1, DH), lambda i: (0, 0)),
            pl.BlockSpec((1, DH), lambda i: (0, 0)),
        ],
        out_specs=[pl.BlockSpec((BM, DH), lambda i: (i, 0)),
                   pl.BlockSpec((NC, BM, ynh), lambda i: (0, i, 0))],
        out_shape=[jax.ShapeDtypeStruct((N_PAD, DH), jnp.float32),
                   jax.ShapeDtypeStruct((NC, N_PAD, ynh), jnp.bfloat16)],
    )(acc, cnt, x, wrT, wnextT, bl, g, beta)


D3 = 64


def _final_body(acc_ref, cnt_ref, h_ref, wr_ref, bl_ref, o_ref):
    cnt = cnt_ref[0, :, 0:1] + cnt_ref[1, :, 0:1]
    inv = 1.0 / jnp.maximum(cnt, 1.0)
    agg = jnp.concatenate([acc_ref[0], acc_ref[1]], axis=1).astype(jnp.float32)
    z = agg * inv + bl_ref[...] + jnp.dot(
        h_ref[...], wr_ref[...], preferred_element_type=jnp.float32)
    col = lax.broadcasted_iota(jnp.int32, z.shape, 1)
    valid = col < DOUT
    zm = jnp.where(valid, z, -jnp.inf)
    m = jnp.max(zm, axis=1, keepdims=True)
    e = jnp.where(valid, jnp.exp(z - m), 0.0)
    lse = jnp.log(jnp.sum(e, axis=1, keepdims=True))
    o_ref[...] = z - m - lse


def _final(acc, cnt, h, wrT, bl):
    return pl.pallas_call(
        _final_body,
        grid=(N_PAD // BM,),
        in_specs=[
            pl.BlockSpec((NC, BM, D3 // 2), lambda i: (0, i, 0)),
            pl.BlockSpec((NC, BM, 16), lambda i: (0, i, 0)),
            pl.BlockSpec((BM, DH), lambda i: (i, 0)),
            pl.BlockSpec((DH, D3), lambda i: (0, 0)),
            pl.BlockSpec((1, D3), lambda i: (0, 0)),
        ],
        out_specs=pl.BlockSpec((BM, D3), lambda i: (i, 0)),
        out_shape=jax.ShapeDtypeStruct((N_PAD, D3), jnp.float32),
    )(acc, cnt, h, wrT, bl)


def kernel(x, edge_index, Wl1, bl1, Wr1, g1, b1, Wl2, bl2, Wr2, g2, b2,
           Wl3, bl3, Wr3):
    e = edge_index.shape[1]
    blk_edges = NS * EB * NBUF
    e_pad = ((e + blk_edges - 1) // blk_edges) * blk_edges
    rb_total = e_pad // EB
    rb_per_worker = rb_total // NS

    src = jnp.concatenate(
        [edge_index[0], jnp.zeros((e_pad - e,), jnp.int32)]).reshape(rb_total, EB)
    dst = jnp.concatenate(
        [edge_index[1], jnp.full((e_pad - e,), N, jnp.int32)]).reshape(rb_total, EB)

    x_p = jnp.concatenate([x, jnp.zeros((N_PAD - N, DIN), jnp.float32)])
    zeros_d = jnp.zeros((ROWS_PER_TILE, DHALF), jnp.bfloat16)
    zeros16 = jnp.zeros((ROWS_PER_TILE, 16), jnp.float32)
    ones16 = jnp.ones((EB, 16), jnp.float32)

    wl3T = jnp.zeros((DH, D3), jnp.float32).at[:, :DOUT].set(Wl3.T)
    wr3T = jnp.zeros((DH, D3), jnp.float32).at[:, :DOUT].set(Wr3.T)
    bl3p = jnp.zeros((1, D3), jnp.float32).at[0, :DOUT].set(bl3)

    agg_cnt = _make_agg(rb_per_worker, True, DHALF)
    agg2 = _make_agg(rb_per_worker, False, DHALF)
    agg3 = _make_agg(rb_per_worker, False, D3 // 2)
    zeros_d3 = jnp.zeros((ROWS_PER_TILE, D3 // 2), jnp.bfloat16)

    r2 = lambda v: v.reshape(1, DH)

    y1 = _matmul(x_p, Wl1.T)
    acc1, cnt = agg_cnt(y1, src, dst, zeros_d, zeros16, ones16)
    h1, y2 = _combine(acc1, cnt, x_p, Wr1.T, Wl2.T, r2(bl1), r2(g1), r2(b1))
    (acc2,) = agg2(y2, src, dst, zeros_d, zeros16, ones16)
    h2, y3 = _combine(acc2, cnt, h1, Wr2.T, wl3T, r2(bl2), r2(g2), r2(b2))
    (acc3,) = agg3(y3, src, dst, zeros_d3, zeros16, ones16)
    o = _final(acc3, cnt, h2, wr3T, bl3p)
    return o[:N, :DOUT]


# speedup vs baseline: 2.1681x; 1.0039x over previous
import jax
import jax.numpy as jnp
from jax import lax
from jax.experimental import pallas as pl
from jax.experimental.pallas import tpu as pltpu
from jax.experimental.pallas import tpu_sc as plsc

N = 10000
DIN = 128
DH = 128
DOUT = 47
EPS = 1e-5

N_PAD = 10240
DHALF = DH // 2
BM = 512
EB = 320
NBUF = 8
PRE = 6
NC = 2
NS = 16
ROWS_PER_TILE = N_PAD // NS


def _make_agg(rb_per_worker: int, with_cnt: bool, dh: int):
    mesh = plsc.VectorSubcoreMesh(core_axis_name="c", subcore_axis_name="s")
    out_type = [jax.ShapeDtypeStruct((NC, N_PAD, dh), jnp.bfloat16)]
    scratch = (
        [pltpu.VMEM((EB,), jnp.int32) for _ in range(NBUF)]
        + [pltpu.VMEM((EB,), jnp.int32) for _ in range(NBUF)]
        + [pltpu.VMEM((EB, dh), jnp.bfloat16) for _ in range(NBUF)]
        + [pltpu.VMEM_SHARED((N_PAD, dh), jnp.bfloat16)]
        + [pltpu.SemaphoreType.DMA for _ in range(3 * NBUF)]
    )
    if with_cnt:
        out_type.append(jax.ShapeDtypeStruct((NC, N_PAD, 16), jnp.float32))
        scratch += [
            pltpu.VMEM((EB, 16), jnp.float32),
            pltpu.VMEM_SHARED((N_PAD, 16), jnp.float32),
        ]

    n_iters = rb_per_worker // NBUF

    def body(y_hbm, src_hbm, dst_hbm, zeros_hbm, zeros16_hbm, ones_hbm,
             acc_out, *rest):
        rest = list(rest)
        cnt_out = rest.pop(0) if with_cnt else None
        srcs = [rest.pop(0) for _ in range(NBUF)]
        dsts = [rest.pop(0) for _ in range(NBUF)]
        bufs = [rest.pop(0) for _ in range(NBUF)]
        acc_sh = rest.pop(0)
        semG = [rest.pop(0) for _ in range(NBUF)]
        semS = [rest.pop(0) for _ in range(NBUF)]
        semI = [rest.pop(0) for _ in range(NBUF)]
        if with_cnt:
            ones_v, cnt_sh = rest
        c = lax.axis_index("c")
        s = lax.axis_index("s")
        row0 = s * ROWS_PER_TILE
        tile_base = s * rb_per_worker

        pltpu.sync_copy(zeros_hbm, acc_sh.at[pl.ds(row0, ROWS_PER_TILE)])
        if with_cnt:
            pltpu.sync_copy(zeros16_hbm, cnt_sh.at[pl.ds(row0, ROWS_PER_TILE)])
            pltpu.sync_copy(ones_hbm, ones_v)
        plsc.subcore_barrier()

        def load_idx(sbuf, dbuf, row, sem):
            return [
                pltpu.async_copy(src_hbm.at[row], sbuf, sem),
                pltpu.async_copy(dst_hbm.at[row], dbuf, sem),
            ]

        def fire_gathers(sbuf, rowbuf, sem):
            return [pltpu.async_copy(y_hbm.at[c].at[sbuf], rowbuf, sem)]

        def fire_scatters(dbuf, rowbuf, sem):
            return [pltpu.async_copy(rowbuf, acc_sh.at[dbuf], sem, add=True)]

        def cnt_adds(dbuf, pred):
            if not with_cnt:
                return

            @pl.when(pred)
            def _():
                pltpu.sync_copy(ones_v, cnt_sh.at[dbuf], add=True)

        def drain(descs):
            for d in descs:
                d.wait()

        @pl.loop(0, n_iters)
        def _(m):
            pred = (c == 0) == (m < n_iters // 2)
            base = tile_base + m * NBUF
            idx = [load_idx(srcs[k], dsts[k], base + k, semI[k])
                   for k in range(NBUF)]
            g = []
            for k in range(PRE):
                drain(idx[k])
                g.append(fire_gathers(srcs[k], bufs[k], semG[k]))
            sc = []
            for k in range(NBUF):
                drain(g[k])
                sc.append(fire_scatters(dsts[k], bufs[k], semS[k]))
                if k + PRE < NBUF:
                    drain(idx[k + PRE])
                    g.append(fire_gathers(srcs[k + PRE], bufs[k + PRE],
                                          semG[k + PRE]))
                cnt_adds(dsts[k], pred)
            for k in range(NBUF):
                drain(sc[k])

        plsc.subcore_barrier()
        sl = pl.ds(row0, ROWS_PER_TILE)
        pltpu.sync_copy(acc_sh.at[sl], acc_out.at[c].at[sl])
        if with_cnt:
            pltpu.sync_copy(cnt_sh.at[sl], cnt_out.at[c].at[sl])

    return pl.kernel(body, out_type=tuple(out_type), mesh=mesh,
                     scratch_types=tuple(scratch),
                     compiler_params=pltpu.CompilerParams(
                         use_tc_tiling_on_sc=False))


def _mm_body(x_ref, w_ref, o_ref):
    y = jnp.dot(x_ref[...], w_ref[...],
                preferred_element_type=jnp.float32).astype(jnp.bfloat16)
    ynh = o_ref.shape[2]
    o_ref[0] = y[:, :ynh]
    o_ref[1] = y[:, ynh:]


def _matmul(x, w):
    n, k = x.shape
    m = w.shape[1]
    return pl.pallas_call(
        _mm_body,
        grid=(n // BM,),
        in_specs=[pl.BlockSpec((BM, k), lambda i: (i, 0)),
                  pl.BlockSpec((k, m), lambda i: (0, 0))],
        out_specs=pl.BlockSpec((NC, BM, m // 2), lambda i: (0, i, 0)),
        out_shape=jax.ShapeDtypeStruct((NC, n, m // 2), jnp.bfloat16),
    )(x, w)


def _combine_body(acc_ref, cnt_ref, x_ref, wr_ref, wn_ref, bl_ref, g_ref,
                  beta_ref, h_ref, y_ref):
    cnt = cnt_ref[0, :, 0:1] + cnt_ref[1, :, 0:1]
    inv = 1.0 / jnp.maximum(cnt, 1.0)
    agg = jnp.concatenate([acc_ref[0], acc_ref[1]], axis=1).astype(jnp.float32)
    h = agg * inv + bl_ref[...] + jnp.dot(
        x_ref[...], wr_ref[...], preferred_element_type=jnp.float32)
    scale = g_ref[...] * (1.0 / jnp.sqrt(1.0 + EPS))
    h = jnp.maximum(scale * h + beta_ref[...], 0.0)
    h_ref[...] = h
    y = jnp.dot(h, wn_ref[...],
                preferred_element_type=jnp.float32).astype(jnp.bfloat16)
    ynh = y_ref.shape[2]
    y_ref[0] = y[:, :ynh]
    y_ref[1] = y[:, ynh:]


def _combine(acc, cnt, x, wrT, wnextT, bl, g, beta):
    ah = acc.shape[2]
    ynh = wnextT.shape[1] // 2
    return pl.pallas_call(
        _combine_body,
        grid=(N_PAD // BM,),
        in_specs=[
            pl.BlockSpec((NC, BM, ah), lambda i: (0, i, 0)),
            pl.BlockSpec((NC, BM, 16), lambda i: (0, i, 0)),
            pl.BlockSpec((BM, DH), lambda i: (i, 0)),
            pl.BlockSpec((DH, DH), lambda i: (0, 0)),
            pl.BlockSpec((DH, wnextT.shape[1]), lambda i: (0, 0)),
            pl.BlockSpec((1, DH), lambda i: (0, 0)),
            pl.BlockSpec((1, DH), lambda i: (0, 0)),
            pl.BlockSpec((1, DH), lambda i: (0, 0)),
        ],
        out_specs=[pl.BlockSpec((BM, DH), lambda i: (i, 0)),
                   pl.BlockSpec((NC, BM, ynh), lambda i: (0, i, 0))],
        out_shape=[jax.ShapeDtypeStruct((N_PAD, DH), jnp.float32),
                   jax.ShapeDtypeStruct((NC, N_PAD, ynh), jnp.bfloat16)],
    )(acc, cnt, x, wrT, wnextT, bl, g, beta)


D3 = 64


def _final_body(acc_ref, cnt_ref, h_ref, wr_ref, bl_ref, o_ref):
    cnt = cnt_ref[0, :, 0:1] + cnt_ref[1, :, 0:1]
    inv = 1.0 / jnp.maximum(cnt, 1.0)
    agg = jnp.concatenate([acc_ref[0], acc_ref[1]], axis=1).astype(jnp.float32)
    z = agg * inv + bl_ref[...] + jnp.dot(
        h_ref[...], wr_ref[...], preferred_element_type=jnp.float32)
    col = lax.broadcasted_iota(jnp.int32, z.shape, 1)
    valid = col < DOUT
    zm = jnp.where(valid, z, -jnp.inf)
    m = jnp.max(zm, axis=1, keepdims=True)
    e = jnp.where(valid, jnp.exp(z - m), 0.0)
    lse = jnp.log(jnp.sum(e, axis=1, keepdims=True))
    o_ref[...] = z - m - lse


def _final(acc, cnt, h, wrT, bl):
    return pl.pallas_call(
        _final_body,
        grid=(N_PAD // BM,),
        in_specs=[
            pl.BlockSpec((NC, BM, D3 // 2), lambda i: (0, i, 0)),
            pl.BlockSpec((NC, BM, 16), lambda i: (0, i, 0)),
            pl.BlockSpec((BM, DH), lambda i: (i, 0)),
            pl.BlockSpec((DH, D3), lambda i: (0, 0)),
            pl.BlockSpec((1, D3), lambda i: (0, 0)),
        ],
        out_specs=pl.BlockSpec((BM, D3), lambda i: (i, 0)),
        out_shape=jax.ShapeDtypeStruct((N_PAD, D3), jnp.float32),
    )(acc, cnt, h, wrT, bl)


def kernel(x, edge_index, Wl1, bl1, Wr1, g1, b1, Wl2, bl2, Wr2, g2, b2,
           Wl3, bl3, Wr3):
    e = edge_index.shape[1]
    blk_edges = NS * EB * NBUF
    e_pad = ((e + blk_edges - 1) // blk_edges) * blk_edges
    rb_total = e_pad // EB
    rb_per_worker = rb_total // NS

    src = jnp.concatenate(
        [edge_index[0], jnp.zeros((e_pad - e,), jnp.int32)]).reshape(rb_total, EB)
    dst = jnp.concatenate(
        [edge_index[1], jnp.full((e_pad - e,), N, jnp.int32)]).reshape(rb_total, EB)

    x_p = jnp.concatenate([x, jnp.zeros((N_PAD - N, DIN), jnp.float32)])
    zeros_d = jnp.zeros((ROWS_PER_TILE, DHALF), jnp.bfloat16)
    zeros16 = jnp.zeros((ROWS_PER_TILE, 16), jnp.float32)
    ones16 = jnp.ones((EB, 16), jnp.float32)

    wl3T = jnp.zeros((DH, D3), jnp.float32).at[:, :DOUT].set(Wl3.T)
    wr3T = jnp.zeros((DH, D3), jnp.float32).at[:, :DOUT].set(Wr3.T)
    bl3p = jnp.zeros((1, D3), jnp.float32).at[0, :DOUT].set(bl3)

    agg_cnt = _make_agg(rb_per_worker, True, DHALF)
    agg2 = _make_agg(rb_per_worker, False, DHALF)
    agg3 = _make_agg(rb_per_worker, False, D3 // 2)
    zeros_d3 = jnp.zeros((ROWS_PER_TILE, D3 // 2), jnp.bfloat16)

    r2 = lambda v: v.reshape(1, DH)

    y1 = _matmul(x_p, Wl1.T)
    acc1, cnt = agg_cnt(y1, src, dst, zeros_d, zeros16, ones16)
    h1, y2 = _combine(acc1, cnt, x_p, Wr1.T, Wl2.T, r2(bl1), r2(g1), r2(b1))
    (acc2,) = agg2(y2, src, dst, zeros_d, zeros16, ones16)
    h2, y3 = _combine(acc2, cnt, h1, Wr2.T, wl3T, r2(bl2), r2(g2), r2(b2))
    (acc3,) = agg3(y3, src, dst, zeros_d3, zeros16, ones16)
    o = _final(acc3, cnt, h2, wr3T, bl3p)
    return o[:N, :DOUT]
